# baseline (speedup 1.0000x reference)
import functools

import jax
import jax.numpy as jnp
from jax import lax
from jax.experimental import pallas as pl
from jax.experimental.pallas import tpu as pltpu
from jax.experimental.pallas import tpu_sc as plsc

EPS = 0.1
ITERS = 10

NC = 2
NS = 16
NW = NC * NS
CHUNK = 80
NB = 4
NSLOT = 8


def _edge_pass(n_rows, c_real, d):
    rows_per_tile = n_rows // NS
    n_full = rows_per_tile // CHUNK
    rem = rows_per_tile - n_full * CHUNK
    mesh = plsc.VectorSubcoreMesh(core_axis_name="c", subcore_axis_name="s")

    @functools.partial(
        pl.kernel,
        mesh=mesh,
        out_type=jax.ShapeDtypeStruct((NC, n_rows, d), jnp.float32),
        scratch_types=[
            pltpu.VMEM((NSLOT, 2, CHUNK), jnp.int32),
            [pltpu.VMEM((CHUNK, d), jnp.float32)] * NB,
            pltpu.VMEM_SHARED((n_rows, d), jnp.float32),
            [pltpu.SemaphoreType.DMA] * NB,
            [pltpu.SemaphoreType.DMA] * NB,
            [pltpu.SemaphoreType.DMA] * NSLOT,
        ],
    )
    def edge_kernel(idx_hbm, g_hbm, t_out, ring, bufs, t_sh, gsems, ssems,
                    sis):
        cid = lax.axis_index("c")
        sid = lax.axis_index("s")

        def idx_start(chunk, slot):
            pltpu.make_async_copy(idx_hbm.at[cid, sid, chunk],
                                  ring.at[slot], sis[slot]).start()

        def idx_wait(chunk, slot):
            pltpu.make_async_copy(idx_hbm.at[cid, sid, chunk],
                                  ring.at[slot], sis[slot]).wait()

        def gather_start(slot, b):
            pltpu.make_async_copy(g_hbm.at[ring.at[slot, 0]], bufs[b],
                                  gsems[b]).start()

        def gather_wait(slot, b):
            pltpu.make_async_copy(g_hbm.at[ring.at[slot, 0]], bufs[b],
                                  gsems[b]).wait()

        def scat_start(slot, b):
            pltpu.async_copy(bufs[b], t_sh.at[ring.at[slot, 1]], ssems[b],
                             add=True)

        def scat_wait(slot, b):
            pltpu.make_async_copy(bufs[b], t_sh.at[ring.at[slot, 1]],
                                  ssems[b]).wait()

        def _zrow(r, carry):
            for k in range(d // 16):
                bufs[0][r, pl.ds(k * 16, 16)] = jnp.zeros((16,), jnp.float32)
            return carry
        lax.fori_loop(0, CHUNK, _zrow, 0)
        base = sid * rows_per_tile
        for c in range(n_full):
            pltpu.sync_copy(bufs[0], t_sh.at[pl.ds(base + c * CHUNK, CHUNK)])
        if rem:
            pltpu.sync_copy(bufs[0].at[pl.ds(0, rem)],
                            t_sh.at[pl.ds(base + n_full * CHUNK, rem)])

        for s in range(5):
            idx_start(s, s)
        idx_wait(0, 0)
        plsc.subcore_barrier()
        gather_start(0, 0)
        idx_wait(1, 1)
        gather_start(1, 1)

        def _step(c, slot, first=False, idx_pf=True, gather_pf=True):
            b = slot % NB
            if not first:
                scat_wait((slot - 2) % NSLOT, (slot + 2) % NB)
            if idx_pf:
                idx_start(c + 5, (slot + 5) % NSLOT)
            if gather_pf:
                idx_wait(c + 2, (slot + 2) % NSLOT)
                gather_start((slot + 2) % NSLOT, (slot + 2) % NB)
            gather_wait(slot, b)
            scat_start(slot, b)

        _step(0, 0, first=True)
        _step(1, 1, first=True)
        for k in range(2, NSLOT):
            _step(k, k)

        def _body(j, carry):
            c0 = NSLOT * j
            for k in range(NSLOT):
                _step(c0 + k, k)
            return carry
        lax.fori_loop(1, c_real // NSLOT - 1, _body, 0)

        c0 = c_real - NSLOT
        for k in range(NSLOT):
            c = c0 + k
            _step(c, k, idx_pf=(c + 5 < c_real), gather_pf=(c + 2 < c_real))

        scat_wait((c_real - 2) % NSLOT, (c_real - 2) % NB)
        scat_wait((c_real - 1) % NSLOT, (c_real - 1) % NB)
        plsc.subcore_barrier()

        pltpu.sync_copy(t_sh.at[pl.ds(base, rows_per_tile)],
                        t_out.at[cid, pl.ds(base, rows_per_tile)])

    return edge_kernel


def _setup_dense(n, d, blk):
    def body(t0_ref, t1_ref, x_ref, dis_ref, g_ref):
        deg = t0_ref[...] + t1_ref[...] + 2.0
        dis = lax.rsqrt(deg)
        dis_ref[...] = dis
        g_ref[...] = dis * x_ref[...]

    return pl.pallas_call(
        body,
        grid=(n // blk,),
        in_specs=[pl.BlockSpec((blk, d), lambda i: (i, 0))] * 3,
        out_specs=[pl.BlockSpec((blk, d), lambda i: (i, 0))] * 2,
        out_shape=[jax.ShapeDtypeStruct((n, d), jnp.float32)] * 2,
    )


def _combine_dense(n, d, blk):
    def body(h_ref, t0_ref, t1_ref, dis_ref, hn_ref, gn_ref):
        dis = dis_ref[...]
        h = h_ref[...]
        hn = (h * (1.0 - (2.0 * EPS) * dis * dis)
              - EPS * dis * (t0_ref[...] + t1_ref[...]))
        hn_ref[...] = hn
        gn_ref[...] = dis * hn

    return pl.pallas_call(
        body,
        grid=(n // blk,),
        in_specs=[pl.BlockSpec((blk, d), lambda i: (i, 0))] * 4,
        out_specs=[pl.BlockSpec((blk, d), lambda i: (i, 0))] * 2,
        out_shape=[jax.ShapeDtypeStruct((n, d), jnp.float32)] * 2,
    )


def _readout(n, d, out_d, blk):
    def body(h_ref, w_ref, b_ref, o_ref):
        ht = jnp.tanh(h_ref[...])
        o_ref[...] = lax.dot_general(
            ht, w_ref[...], (((1,), (1,)), ((), ())),
            preferred_element_type=jnp.float32) + b_ref[...]

    return pl.pallas_call(
        body,
        grid=(n // blk,),
        in_specs=[
            pl.BlockSpec((blk, d), lambda i: (i, 0)),
            pl.BlockSpec((out_d, d), lambda i: (0, 0)),
            pl.BlockSpec((1, out_d), lambda i: (0, 0)),
        ],
        out_specs=pl.BlockSpec((blk, out_d), lambda i: (i, 0)),
        out_shape=jax.ShapeDtypeStruct((n, out_d), jnp.float32),
    )


def kernel(x, edge_index, W, b):
    n, d = x.shape
    e = edge_index.shape[1]
    out_d = W.shape[0]

    per_tile = -(-e // NW)
    c_real = -(-per_tile // CHUNK)
    c_real = -(-c_real // NSLOT) * NSLOT
    slots = NW * c_real * CHUNK
    n_rows = -(-(n + 1) // (NS * 8)) * (NS * 8)

    order = jnp.argsort(edge_index[0])
    src = edge_index[0][order]
    dst = edge_index[1][order]
    src_p = jnp.concatenate([src, jnp.full((slots - e,), n - 1, jnp.int32)])
    dst_p = jnp.concatenate([dst, jnp.full((slots - e,), n, jnp.int32)])
    idx = jnp.stack([src_p.reshape(NC, NS, c_real, CHUNK),
                     dst_p.reshape(NC, NS, c_real, CHUNK)], axis=3)

    edge_sc = _edge_pass(n_rows, c_real, d)
    blk = 2000
    setup_tc = _setup_dense(n, d, blk)
    combine_tc = _combine_dense(n, d, blk)
    readout_tc = _readout(n, d, out_d, blk)

    ones = jnp.ones((n, d), jnp.float32)
    t = edge_sc(idx, ones)
    dis, g = setup_tc(t[0, :n], t[1, :n], x)
    h = x
    for _ in range(ITERS):
        t = edge_sc(idx, g)
        h, g = combine_tc(h, t[0, :n], t[1, :n], dis)
    return readout_tc(h, W, b.reshape(1, out_d))


# speedup vs baseline: 1.0263x; 1.0263x over previous
import functools

import jax
import jax.numpy as jnp
from jax import lax
from jax.experimental import pallas as pl
from jax.experimental.pallas import tpu as pltpu
from jax.experimental.pallas import tpu_sc as plsc

EPS = 0.1
ITERS = 10

NC = 2
NS = 16
NW = NC * NS
CHUNK = 64
NB = 5
NSLOT = 10


def _edge_pass(n_rows, c_real, d):
    rows_per_tile = n_rows // NS
    n_full = rows_per_tile // CHUNK
    rem = rows_per_tile - n_full * CHUNK
    mesh = plsc.VectorSubcoreMesh(core_axis_name="c", subcore_axis_name="s")

    @functools.partial(
        pl.kernel,
        mesh=mesh,
        out_type=jax.ShapeDtypeStruct((NC, n_rows, d), jnp.float32),
        scratch_types=[
            pltpu.VMEM((NSLOT, 2, CHUNK), jnp.int32),
            [pltpu.VMEM((CHUNK, d), jnp.float32)] * NB,
            pltpu.VMEM_SHARED((n_rows, d), jnp.float32),
            [pltpu.SemaphoreType.DMA] * NB,
            [pltpu.SemaphoreType.DMA] * NB,
            [pltpu.SemaphoreType.DMA] * NSLOT,
        ],
    )
    def edge_kernel(idx_hbm, g_hbm, t_out, ring, bufs, t_sh, gsems, ssems,
                    sis):
        cid = lax.axis_index("c")
        sid = lax.axis_index("s")

        def idx_start(chunk, slot):
            pltpu.make_async_copy(idx_hbm.at[cid, sid, chunk],
                                  ring.at[slot], sis[slot]).start()

        def idx_wait(chunk, slot):
            pltpu.make_async_copy(idx_hbm.at[cid, sid, chunk],
                                  ring.at[slot], sis[slot]).wait()

        def gather_start(slot, b):
            pltpu.make_async_copy(g_hbm.at[ring.at[slot, 0]], bufs[b],
                                  gsems[b]).start()

        def gather_wait(slot, b):
            pltpu.make_async_copy(g_hbm.at[ring.at[slot, 0]], bufs[b],
                                  gsems[b]).wait()

        def scat_start(slot, b):
            pltpu.async_copy(bufs[b], t_sh.at[ring.at[slot, 1]], ssems[b],
                             add=True)

        def scat_wait(slot, b):
            pltpu.make_async_copy(bufs[b], t_sh.at[ring.at[slot, 1]],
                                  ssems[b]).wait()

        def _zrow(r, carry):
            for k in range(d // 16):
                bufs[0][r, pl.ds(k * 16, 16)] = jnp.zeros((16,), jnp.float32)
            return carry
        lax.fori_loop(0, CHUNK, _zrow, 0)
        base = sid * rows_per_tile
        for c in range(n_full):
            pltpu.sync_copy(bufs[0], t_sh.at[pl.ds(base + c * CHUNK, CHUNK)])
        if rem:
            pltpu.sync_copy(bufs[0].at[pl.ds(0, rem)],
                            t_sh.at[pl.ds(base + n_full * CHUNK, rem)])

        for s in range(NSLOT - 2):
            idx_start(s, s)
        idx_wait(0, 0)
        plsc.subcore_barrier()
        gather_start(0, 0)
        idx_wait(1, 1)
        gather_start(1, 1)
        idx_wait(2, 2)
        gather_start(2, 2)

        def _step(c, slot, first=False, idx_pf=True, gather_pf=True):
            b = slot % NB
            if not first:
                scat_wait((slot - 2) % NSLOT, (slot - 2) % NB)
            if idx_pf:
                idx_start(c + 8, (slot + 8) % NSLOT)
            if gather_pf:
                idx_wait(c + 3, (slot + 3) % NSLOT)
                gather_start((slot + 3) % NSLOT, (slot + 3) % NB)
            gather_wait(slot, b)
            scat_start(slot, b)

        _step(0, 0, first=True)
        _step(1, 1, first=True)
        for k in range(2, NSLOT):
            _step(k, k)

        def _body(j, carry):
            c0 = NSLOT * j
            for k in range(NSLOT):
                _step(c0 + k, k)
            return carry
        lax.fori_loop(1, c_real // NSLOT - 1, _body, 0)

        c0 = c_real - NSLOT
        for k in range(NSLOT):
            c = c0 + k
            _step(c, k, idx_pf=(c + 8 < c_real), gather_pf=(c + 3 < c_real))

        scat_wait((c_real - 2) % NSLOT, (c_real - 2) % NB)
        scat_wait((c_real - 1) % NSLOT, (c_real - 1) % NB)
        plsc.subcore_barrier()

        pltpu.sync_copy(t_sh.at[pl.ds(base, rows_per_tile)],
                        t_out.at[cid, pl.ds(base, rows_per_tile)])

    return edge_kernel


def _setup_dense(n, d, blk):
    def body(t0_ref, t1_ref, x_ref, dis_ref, g_ref):
        deg = t0_ref[...] + t1_ref[...] + 2.0
        dis = lax.rsqrt(deg)
        dis_ref[...] = dis
        g_ref[...] = dis * x_ref[...]

    return pl.pallas_call(
        body,
        grid=(n // blk,),
        in_specs=[pl.BlockSpec((blk, d), lambda i: (i, 0))] * 3,
        out_specs=[pl.BlockSpec((blk, d), lambda i: (i, 0))] * 2,
        out_shape=[jax.ShapeDtypeStruct((n, d), jnp.float32)] * 2,
    )


def _combine_dense(n, d, blk):
    def body(h_ref, t0_ref, t1_ref, dis_ref, hn_ref, gn_ref):
        dis = dis_ref[...]
        h = h_ref[...]
        hn = (h * (1.0 - (2.0 * EPS) * dis * dis)
              - EPS * dis * (t0_ref[...] + t1_ref[...]))
        hn_ref[...] = hn
        gn_ref[...] = dis * hn

    return pl.pallas_call(
        body,
        grid=(n // blk,),
        in_specs=[pl.BlockSpec((blk, d), lambda i: (i, 0))] * 4,
        out_specs=[pl.BlockSpec((blk, d), lambda i: (i, 0))] * 2,
        out_shape=[jax.ShapeDtypeStruct((n, d), jnp.float32)] * 2,
    )


def _readout(n, d, out_d, blk):
    def body(h_ref, w_ref, b_ref, o_ref):
        ht = jnp.tanh(h_ref[...])
        o_ref[...] = lax.dot_general(
            ht, w_ref[...], (((1,), (1,)), ((), ())),
            preferred_element_type=jnp.float32) + b_ref[...]

    return pl.pallas_call(
        body,
        grid=(n // blk,),
        in_specs=[
            pl.BlockSpec((blk, d), lambda i: (i, 0)),
            pl.BlockSpec((out_d, d), lambda i: (0, 0)),
            pl.BlockSpec((1, out_d), lambda i: (0, 0)),
        ],
        out_specs=pl.BlockSpec((blk, out_d), lambda i: (i, 0)),
        out_shape=jax.ShapeDtypeStruct((n, out_d), jnp.float32),
    )


def kernel(x, edge_index, W, b):
    n, d = x.shape
    e = edge_index.shape[1]
    out_d = W.shape[0]

    per_tile = -(-e // NW)
    c_real = -(-per_tile // CHUNK)
    c_real = -(-c_real // NSLOT) * NSLOT
    slots = NW * c_real * CHUNK
    n_rows = -(-(n + 1) // (NS * 8)) * (NS * 8)

    src = edge_index[0]
    dst = edge_index[1]
    src_p = jnp.concatenate([src, jnp.zeros((slots - e,), jnp.int32)])
    dst_p = jnp.concatenate([dst, jnp.full((slots - e,), n, jnp.int32)])
    idx = jnp.stack([src_p.reshape(NC, NS, c_real, CHUNK),
                     dst_p.reshape(NC, NS, c_real, CHUNK)], axis=3)

    edge_sc = _edge_pass(n_rows, c_real, d)
    blk = 2000
    setup_tc = _setup_dense(n, d, blk)
    combine_tc = _combine_dense(n, d, blk)
    readout_tc = _readout(n, d, out_d, blk)

    ones = jnp.ones((n, d), jnp.float32)
    t = edge_sc(idx, ones)
    dis, g = setup_tc(t[0, :n], t[1, :n], x)
    h = x
    for _ in range(ITERS):
        t = edge_sc(idx, g)
        h, g = combine_tc(h, t[0, :n], t[1, :n], dis)
    return readout_tc(h, W, b.reshape(1, out_d))


# speedup vs baseline: 1.3569x; 1.3221x over previous
import functools

import jax
import jax.numpy as jnp
from jax import lax
from jax.experimental import pallas as pl
from jax.experimental.pallas import tpu as pltpu
from jax.experimental.pallas import tpu_sc as plsc

EPS = 0.1
ITERS = 10

NC = 2
NS = 16
NW = NC * NS
CHUNK = 80
NB = 4
NSLOT = 8


def _edge_pass(n_rows, c_real, d):
    rows_per_tile = n_rows // NS
    n_full = rows_per_tile // CHUNK
    rem = rows_per_tile - n_full * CHUNK
    mesh = plsc.VectorSubcoreMesh(core_axis_name="c", subcore_axis_name="s")

    @functools.partial(
        pl.kernel,
        mesh=mesh,
        compiler_params=pltpu.CompilerParams(use_tc_tiling_on_sc=False),
        out_type=jax.ShapeDtypeStruct((NC, n_rows, d), jnp.float32),
        scratch_types=[
            pltpu.VMEM((NSLOT, 2, CHUNK), jnp.int32),
            [pltpu.VMEM((CHUNK, d), jnp.float32)] * NB,
            pltpu.VMEM_SHARED((n_rows, d), jnp.float32),
            [pltpu.SemaphoreType.DMA] * NB,
            [pltpu.SemaphoreType.DMA] * NB,
            [pltpu.SemaphoreType.DMA] * NSLOT,
        ],
    )
    def edge_kernel(idx_hbm, g_hbm, t_out, ring, bufs, t_sh, gsems, ssems,
                    sis):
        cid = lax.axis_index("c")
        sid = lax.axis_index("s")

        def idx_start(chunk, slot):
            pltpu.make_async_copy(idx_hbm.at[cid, sid, chunk],
                                  ring.at[slot], sis[slot]).start()

        def idx_wait(chunk, slot):
            pltpu.make_async_copy(idx_hbm.at[cid, sid, chunk],
                                  ring.at[slot], sis[slot]).wait()

        def gather_start(slot, b):
            pltpu.make_async_copy(g_hbm.at[ring.at[slot, 0]], bufs[b],
                                  gsems[b]).start()

        def gather_wait(slot, b):
            pltpu.make_async_copy(g_hbm.at[ring.at[slot, 0]], bufs[b],
                                  gsems[b]).wait()

        def scat_start(slot, b):
            pltpu.async_copy(bufs[b], t_sh.at[ring.at[slot, 1]], ssems[b],
                             add=True)

        def scat_wait(slot, b):
            pltpu.make_async_copy(bufs[b], t_sh.at[ring.at[slot, 1]],
                                  ssems[b]).wait()

        def _zrow(r, carry):
            for k in range(d // 16):
                bufs[0][r, pl.ds(k * 16, 16)] = jnp.zeros((16,), jnp.float32)
            return carry
        lax.fori_loop(0, CHUNK, _zrow, 0)
        base = sid * rows_per_tile
        for c in range(n_full):
            pltpu.sync_copy(bufs[0], t_sh.at[pl.ds(base + c * CHUNK, CHUNK)])
        if rem:
            pltpu.sync_copy(bufs[0].at[pl.ds(0, rem)],
                            t_sh.at[pl.ds(base + n_full * CHUNK, rem)])

        for s in range(5):
            idx_start(s, s)
        idx_wait(0, 0)
        plsc.subcore_barrier()
        gather_start(0, 0)
        idx_wait(1, 1)
        gather_start(1, 1)

        def _step(c, slot, first=False, idx_pf=True, gather_pf=True):
            b = slot % NB
            if not first:
                scat_wait((slot - 2) % NSLOT, (slot + 2) % NB)
            if idx_pf:
                idx_start(c + 5, (slot + 5) % NSLOT)
            if gather_pf:
                idx_wait(c + 2, (slot + 2) % NSLOT)
                gather_start((slot + 2) % NSLOT, (slot + 2) % NB)
            gather_wait(slot, b)
            scat_start(slot, b)

        _step(0, 0, first=True)
        _step(1, 1, first=True)
        for k in range(2, NSLOT):
            _step(k, k)

        def _body(j, carry):
            c0 = NSLOT * j
            for k in range(NSLOT):
                _step(c0 + k, k)
            return carry
        lax.fori_loop(1, c_real // NSLOT - 1, _body, 0)

        c0 = c_real - NSLOT
        for k in range(NSLOT):
            c = c0 + k
            _step(c, k, idx_pf=(c + 5 < c_real), gather_pf=(c + 2 < c_real))

        scat_wait((c_real - 2) % NSLOT, (c_real - 2) % NB)
        scat_wait((c_real - 1) % NSLOT, (c_real - 1) % NB)
        plsc.subcore_barrier()

        pltpu.sync_copy(t_sh.at[pl.ds(base, rows_per_tile)],
                        t_out.at[cid, pl.ds(base, rows_per_tile)])

    return edge_kernel


def _setup_dense(n, d, blk):
    def body(t0_ref, t1_ref, x_ref, dis_ref, g_ref):
        deg = t0_ref[...] + t1_ref[...] + 2.0
        dis = lax.rsqrt(deg)
        dis_ref[...] = dis
        g_ref[...] = dis * x_ref[...]

    return pl.pallas_call(
        body,
        grid=(n // blk,),
        in_specs=[pl.BlockSpec((blk, d), lambda i: (i, 0))] * 3,
        out_specs=[pl.BlockSpec((blk, d), lambda i: (i, 0))] * 2,
        out_shape=[jax.ShapeDtypeStruct((n, d), jnp.float32)] * 2,
    )


def _combine_dense(n, d, blk):
    def body(h_ref, t0_ref, t1_ref, dis_ref, hn_ref, gn_ref):
        dis = dis_ref[...]
        h = h_ref[...]
        hn = (h * (1.0 - (2.0 * EPS) * dis * dis)
              - EPS * dis * (t0_ref[...] + t1_ref[...]))
        hn_ref[...] = hn
        gn_ref[...] = dis * hn

    return pl.pallas_call(
        body,
        grid=(n // blk,),
        in_specs=[pl.BlockSpec((blk, d), lambda i: (i, 0))] * 4,
        out_specs=[pl.BlockSpec((blk, d), lambda i: (i, 0))] * 2,
        out_shape=[jax.ShapeDtypeStruct((n, d), jnp.float32)] * 2,
    )


def _readout(n, d, out_d, blk):
    def body(h_ref, w_ref, b_ref, o_ref):
        ht = jnp.tanh(h_ref[...])
        o_ref[...] = lax.dot_general(
            ht, w_ref[...], (((1,), (1,)), ((), ())),
            preferred_element_type=jnp.float32) + b_ref[...]

    return pl.pallas_call(
        body,
        grid=(n // blk,),
        in_specs=[
            pl.BlockSpec((blk, d), lambda i: (i, 0)),
            pl.BlockSpec((out_d, d), lambda i: (0, 0)),
            pl.BlockSpec((1, out_d), lambda i: (0, 0)),
        ],
        out_specs=pl.BlockSpec((blk, out_d), lambda i: (i, 0)),
        out_shape=jax.ShapeDtypeStruct((n, out_d), jnp.float32),
    )


def kernel(x, edge_index, W, b):
    n, d = x.shape
    e = edge_index.shape[1]
    out_d = W.shape[0]

    per_tile = -(-e // NW)
    c_real = -(-per_tile // CHUNK)
    c_real = -(-c_real // NSLOT) * NSLOT
    slots = NW * c_real * CHUNK
    n_rows = -(-(n + 1) // (NS * 8)) * (NS * 8)

    src = edge_index[0]
    dst = edge_index[1]
    src_p = jnp.concatenate([src, jnp.zeros((slots - e,), jnp.int32)])
    dst_p = jnp.concatenate([dst, jnp.full((slots - e,), n, jnp.int32)])
    idx = jnp.stack([src_p.reshape(NC, NS, c_real, CHUNK),
                     dst_p.reshape(NC, NS, c_real, CHUNK)], axis=3)

    edge_sc = _edge_pass(n_rows, c_real, d)
    blk = 2000
    setup_tc = _setup_dense(n, d, blk)
    combine_tc = _combine_dense(n, d, blk)
    readout_tc = _readout(n, d, out_d, blk)

    ones = jnp.ones((n, d), jnp.float32)
    t = edge_sc(idx, ones)
    dis, g = setup_tc(t[0, :n], t[1, :n], x)
    h = x
    for _ in range(ITERS):
        t = edge_sc(idx, g)
        h, g = combine_tc(h, t[0, :n], t[1, :n], dis)
    return readout_tc(h, W, b.reshape(1, out_d))


# speedup vs baseline: 2.3474x; 1.7300x over previous
import functools

import jax
import jax.numpy as jnp
from jax import lax
from jax.experimental import pallas as pl
from jax.experimental.pallas import tpu as pltpu
from jax.experimental.pallas import tpu_sc as plsc

EPS = 0.1
ITERS = 10

NC = 2
NS = 16
NW = NC * NS
CHUNK = 84
NB = 4
NSLOT = 8


def _edge_pass(n_rows, c_real, d):
    rows_per_tile = n_rows // NS
    n_full = rows_per_tile // CHUNK
    rem = rows_per_tile - n_full * CHUNK
    mesh = plsc.VectorSubcoreMesh(core_axis_name="c", subcore_axis_name="s")

    @functools.partial(
        pl.kernel,
        mesh=mesh,
        compiler_params=pltpu.CompilerParams(use_tc_tiling_on_sc=False),
        out_type=jax.ShapeDtypeStruct((NC, n_rows, d), jnp.float32),
        scratch_types=[
            pltpu.VMEM((NSLOT, 2, CHUNK), jnp.int32),
            [pltpu.VMEM((CHUNK, d), jnp.float32)] * NB,
            pltpu.VMEM_SHARED((n_rows, d), jnp.float32),
            [pltpu.SemaphoreType.DMA] * NB,
            [pltpu.SemaphoreType.DMA] * NB,
            [pltpu.SemaphoreType.DMA] * NSLOT,
        ],
    )
    def edge_kernel(idx_hbm, g_hbm, t_out, ring, bufs, t_sh, gsems, ssems,
                    sis):
        cid = lax.axis_index("c")
        sid = lax.axis_index("s")

        def idx_start(chunk, slot):
            pltpu.make_async_copy(idx_hbm.at[cid, sid, chunk],
                                  ring.at[slot], sis[slot]).start()

        def idx_wait(chunk, slot):
            pltpu.make_async_copy(idx_hbm.at[cid, sid, chunk],
                                  ring.at[slot], sis[slot]).wait()

        def gather_start(slot, b):
            pltpu.make_async_copy(g_hbm.at[ring.at[slot, 0]], bufs[b],
                                  gsems[b]).start()

        def gather_wait(slot, b):
            pltpu.make_async_copy(g_hbm.at[ring.at[slot, 0]], bufs[b],
                                  gsems[b]).wait()

        def scat_start(slot, b):
            pltpu.async_copy(bufs[b], t_sh.at[ring.at[slot, 1]], ssems[b],
                             add=True)

        def scat_wait(slot, b):
            pltpu.make_async_copy(bufs[b], t_sh.at[ring.at[slot, 1]],
                                  ssems[b]).wait()

        def _zrow(r, carry):
            for k in range(d // 16):
                bufs[0][r, pl.ds(k * 16, 16)] = jnp.zeros((16,), jnp.float32)
            return carry
        lax.fori_loop(0, CHUNK, _zrow, 0)
        base = sid * rows_per_tile
        for c in range(n_full):
            pltpu.sync_copy(bufs[0], t_sh.at[pl.ds(base + c * CHUNK, CHUNK)])
        if rem:
            pltpu.sync_copy(bufs[0].at[pl.ds(0, rem)],
                            t_sh.at[pl.ds(base + n_full * CHUNK, rem)])

        for s in range(5):
            idx_start(s, s)
        idx_wait(0, 0)
        plsc.subcore_barrier()
        gather_start(0, 0)
        idx_wait(1, 1)
        gather_start(1, 1)

        def _step(c, slot, first=False, idx_pf=True, gather_pf=True):
            b = slot % NB
            if not first:
                scat_wait((slot - 2) % NSLOT, (slot + 2) % NB)
            if idx_pf:
                idx_start(c + 5, (slot + 5) % NSLOT)
            if gather_pf:
                idx_wait(c + 2, (slot + 2) % NSLOT)
                gather_start((slot + 2) % NSLOT, (slot + 2) % NB)
            gather_wait(slot, b)
            scat_start(slot, b)

        _step(0, 0, first=True)
        _step(1, 1, first=True)
        for k in range(2, NSLOT):
            _step(k, k)

        def _body(j, carry):
            c0 = NSLOT * j
            for k in range(NSLOT):
                _step(c0 + k, k)
            return carry
        lax.fori_loop(1, c_real // NSLOT - 1, _body, 0)

        c0 = c_real - NSLOT
        for k in range(NSLOT):
            c = c0 + k
            _step(c, k, idx_pf=(c + 5 < c_real), gather_pf=(c + 2 < c_real))

        scat_wait((c_real - 2) % NSLOT, (c_real - 2) % NB)
        scat_wait((c_real - 1) % NSLOT, (c_real - 1) % NB)
        plsc.subcore_barrier()

        pltpu.sync_copy(t_sh.at[pl.ds(base, rows_per_tile)],
                        t_out.at[cid, pl.ds(base, rows_per_tile)])

    return edge_kernel


def _setup_dense(n, d, blk):
    def body(t0_ref, t1_ref, x_ref, dis_ref, g_ref):
        deg = t0_ref[...] + t1_ref[...] + 2.0
        dis = lax.rsqrt(deg)
        dis_ref[...] = dis
        g_ref[...] = dis * x_ref[...]

    return pl.pallas_call(
        body,
        grid=(n // blk,),
        in_specs=[pl.BlockSpec((blk, d), lambda i: (i, 0))] * 3,
        out_specs=[pl.BlockSpec((blk, d), lambda i: (i, 0))] * 2,
        out_shape=[jax.ShapeDtypeStruct((n, d), jnp.float32)] * 2,
    )


def _combine_dense(n, d, blk):
    def body(h_ref, t0_ref, t1_ref, dis_ref, hn_ref, gn_ref):
        dis = dis_ref[...]
        h = h_ref[...]
        hn = (h * (1.0 - (2.0 * EPS) * dis * dis)
              - EPS * dis * (t0_ref[...] + t1_ref[...]))
        hn_ref[...] = hn
        gn_ref[...] = dis * hn

    return pl.pallas_call(
        body,
        grid=(n // blk,),
        in_specs=[pl.BlockSpec((blk, d), lambda i: (i, 0))] * 4,
        out_specs=[pl.BlockSpec((blk, d), lambda i: (i, 0))] * 2,
        out_shape=[jax.ShapeDtypeStruct((n, d), jnp.float32)] * 2,
    )


def _readout(n, d, out_d, blk):
    def body(h_ref, w_ref, b_ref, o_ref):
        ht = jnp.tanh(h_ref[...])
        o_ref[...] = lax.dot_general(
            ht, w_ref[...], (((1,), (1,)), ((), ())),
            preferred_element_type=jnp.float32) + b_ref[...]

    return pl.pallas_call(
        body,
        grid=(n // blk,),
        in_specs=[
            pl.BlockSpec((blk, d), lambda i: (i, 0)),
            pl.BlockSpec((out_d, d), lambda i: (0, 0)),
            pl.BlockSpec((1, out_d), lambda i: (0, 0)),
        ],
        out_specs=pl.BlockSpec((blk, out_d), lambda i: (i, 0)),
        out_shape=jax.ShapeDtypeStruct((n, out_d), jnp.float32),
    )


def kernel(x, edge_index, W, b):
    n, d = x.shape
    e = edge_index.shape[1]
    out_d = W.shape[0]

    per_tile = -(-e // NW)
    c_real = -(-per_tile // CHUNK)
    c_real = -(-c_real // NSLOT) * NSLOT
    slots = NW * c_real * CHUNK
    n_rows = -(-(n + 1) // (NS * 8)) * (NS * 8)

    src = edge_index[0]
    dst = edge_index[1]
    src_p = jnp.concatenate([src, jnp.zeros((slots - e,), jnp.int32)])
    dst_p = jnp.concatenate([dst, jnp.full((slots - e,), n, jnp.int32)])
    idx = jnp.stack([src_p.reshape(NC, NS, c_real, CHUNK),
                     dst_p.reshape(NC, NS, c_real, CHUNK)], axis=3)

    edge_sc = _edge_pass(n_rows, c_real, d)
    blk = 2000
    setup_tc = _setup_dense(n, d, blk)
    combine_tc = _combine_dense(n, d, blk)
    readout_tc = _readout(n, d, out_d, blk)

    ones = jnp.ones((n, d), jnp.float32)
    t = edge_sc(idx, ones)
    dis, g = setup_tc(t[0, :n], t[1, :n], x)
    h = x
    for _ in range(ITERS):
        t = edge_sc(idx, g)
        h, g = combine_tc(h, t[0, :n], t[1, :n], dis)
    return readout_tc(h, W, b.reshape(1, out_d))
